# Initial kernel scaffold; baseline (speedup 1.0000x reference)
import jax
import jax.numpy as jnp
from jax.experimental import pallas as pl


def kernel(pos, vel, field, index, edge_index, data_id, a, We0, We1, We2, We3, We4, be0, be1, be2, be3, be4, Wp0, Wp1, Wp2, bp0, bp1, bp2):
    raise NotImplementedError("write your pallas kernel here")


# speedup vs baseline: 6.3607x; 6.3607x over previous
import functools

import jax
import jax.numpy as jnp
from jax import lax
from jax.experimental import pallas as pl
from jax.experimental.pallas import tpu as pltpu
from jax.experimental.pallas import tpu_sc as plsc

_MAX_RADIUS = 0.1
_VNORM = 1.0
_NC = 2
_NS = 16
_NW = _NC * _NS
_TABW = 16
_GB = 1000
_CB = 1000
_BE = 1024
_BN = 1000


def _sc_gather(tab, dst, src, *, interpret=False):
    N, W = tab.shape
    E = dst.shape[0]
    ew = E // _NW
    nchunks = ew // _GB
    mesh = plsc.VectorSubcoreMesh(core_axis_name="c", subcore_axis_name="s")

    @functools.partial(
        pl.kernel,
        out_type=(jax.ShapeDtypeStruct((E, W), jnp.float32),
                  jax.ShapeDtypeStruct((E, W), jnp.float32)),
        mesh=mesh,
        scratch_types=[pltpu.VMEM((_GB,), jnp.int32),
                       pltpu.VMEM((_GB, W), jnp.float32),
                       pltpu.SemaphoreType.DMA],
        compiler_params=pltpu.CompilerParams(use_tc_tiling_on_sc=False),
        interpret=interpret,
    )
    def gk(tab_hbm, dst_hbm, src_hbm, gd_hbm, gs_hbm, idx_v, rows_v, sem):
        wid = lax.axis_index("s") * _NC + lax.axis_index("c")

        def body(t, carry):
            base = wid * ew + t * _GB
            pltpu.sync_copy(dst_hbm.at[pl.ds(base, _GB)], idx_v)
            pltpu.async_copy(tab_hbm.at[idx_v], rows_v, sem).wait()
            pltpu.sync_copy(rows_v, gd_hbm.at[pl.ds(base, _GB), :])
            pltpu.sync_copy(src_hbm.at[pl.ds(base, _GB)], idx_v)
            pltpu.async_copy(tab_hbm.at[idx_v], rows_v, sem).wait()
            pltpu.sync_copy(rows_v, gs_hbm.at[pl.ds(base, _GB), :])
            return carry

        lax.fori_loop(0, nchunks, body, 0)

    return gk(tab, dst, src)


def _sc_scatter(msg, dst, zero_tab, *, interpret=False):
    N, W = zero_tab.shape
    E = dst.shape[0]
    ew = E // _NW
    nchunks = ew // _CB
    rps = N // _NS
    mesh = plsc.VectorSubcoreMesh(core_axis_name="c", subcore_axis_name="s")

    @functools.partial(
        pl.kernel,
        out_type=(jax.ShapeDtypeStruct((N, W), jnp.float32),
                  jax.ShapeDtypeStruct((N, W), jnp.float32)),
        mesh=mesh,
        scratch_types=[pltpu.VMEM((_CB,), jnp.int32),
                       pltpu.VMEM((_CB, W), jnp.float32),
                       pltpu.VMEM_SHARED((N, W), jnp.float32)],
        compiler_params=pltpu.CompilerParams(use_tc_tiling_on_sc=False),
        interpret=interpret,
    )
    def sk(msg_hbm, dst_hbm, zero_hbm, a0_hbm, a1_hbm, idx_v, msg_v, acc_sh):
        cid = lax.axis_index("c")
        sid = lax.axis_index("s")
        wid = sid * _NC + cid

        @pl.when(sid == 0)
        def _zero():
            pltpu.sync_copy(zero_hbm, acc_sh)

        plsc.subcore_barrier()

        def body(t, carry):
            base = wid * ew + t * _CB
            pltpu.sync_copy(dst_hbm.at[pl.ds(base, _CB)], idx_v)
            pltpu.sync_copy(msg_hbm.at[pl.ds(base, _CB), :], msg_v)
            pltpu.sync_copy(msg_v, acc_sh.at[idx_v], add=True)
            return carry

        lax.fori_loop(0, nchunks, body, 0)
        plsc.subcore_barrier()
        row0 = sid * rps

        @pl.when(cid == 0)
        def _w0():
            pltpu.sync_copy(acc_sh.at[pl.ds(row0, rps), :],
                            a0_hbm.at[pl.ds(row0, rps), :])

        @pl.when(cid == 1)
        def _w1():
            pltpu.sync_copy(acc_sh.at[pl.ds(row0, rps), :],
                            a1_hbm.at[pl.ds(row0, rps), :])

    return sk(msg, dst, zero_tab)


def _tc_edge_mlp(gd, gs, w0p, w1, w2, w3, w4p, b0, b1, b2, b3, b4p,
                 *, interpret=False):
    E, W = gd.shape
    H = w1.shape[0]

    def body(gd_ref, gs_ref, w0_ref, w1_ref, w2_ref, w3_ref, w4_ref,
             b0_ref, b1_ref, b2_ref, b3_ref, b4_ref, out_ref):
        gd_b = gd_ref[...]
        gs_b = gs_ref[...]
        diff = gs_b[:, 0:2] - gd_b[:, 0:2]
        r = jnp.sqrt(jnp.sum(diff * diff, axis=1, keepdims=True) + 1e-12)
        scale = 1.0 / _MAX_RADIUS
        feats = jnp.concatenate(
            [diff * scale, r * scale, gd_b[:, 2:4], gs_b[:, 2:4],
             gd_b[:, 4:6], jnp.zeros((gd_b.shape[0], 7), jnp.float32)],
            axis=1)
        h = jnp.dot(feats, w0_ref[...], preferred_element_type=jnp.float32)
        h = jnp.maximum(h + b0_ref[...], 0.0)
        h = jnp.dot(h, w1_ref[...], preferred_element_type=jnp.float32)
        h = jnp.maximum(h + b1_ref[...], 0.0)
        h = jnp.dot(h, w2_ref[...], preferred_element_type=jnp.float32)
        h = jnp.maximum(h + b2_ref[...], 0.0)
        h = jnp.dot(h, w3_ref[...], preferred_element_type=jnp.float32)
        h = jnp.maximum(h + b3_ref[...], 0.0)
        m = jnp.dot(h, w4_ref[...], preferred_element_type=jnp.float32)
        m = m + b4_ref[...]
        out_ref[...] = m[:, :_TABW]

    full = lambda shape: pl.BlockSpec(shape, lambda i: (0, 0))
    return pl.pallas_call(
        body,
        grid=(E // _BE,),
        in_specs=[
            pl.BlockSpec((_BE, W), lambda i: (i, 0)),
            pl.BlockSpec((_BE, W), lambda i: (i, 0)),
            full(w0p.shape), full(w1.shape), full(w2.shape), full(w3.shape),
            full(w4p.shape),
            full(b0.shape), full(b1.shape), full(b2.shape), full(b3.shape),
            full(b4p.shape),
        ],
        out_specs=pl.BlockSpec((_BE, _TABW), lambda i: (i, 0)),
        out_shape=jax.ShapeDtypeStruct((E, _TABW), jnp.float32),
        interpret=interpret,
    )(gd, gs, w0p, w1, w2, w3, w4p, b0, b1, b2, b3, b4p)


def _tc_node_mlp(a0, a1, tab, p0p, p1, p2p, q0, q1, q2p, *, interpret=False):
    N, W = tab.shape

    def body(a0_ref, a1_ref, tab_ref, p0_ref, p1_ref, p2_ref,
             q0_ref, q1_ref, q2_ref, out_ref):
        agg = a0_ref[...][:, 0:2] + a1_ref[...][:, 0:2]
        t = tab_ref[...]
        feats = jnp.concatenate(
            [agg, t[:, 4:6], t[:, 2:4],
             jnp.zeros((agg.shape[0], 10), jnp.float32)], axis=1)
        h = jnp.dot(feats, p0_ref[...], preferred_element_type=jnp.float32)
        h = jnp.maximum(h + q0_ref[...], 0.0)
        h = jnp.dot(h, p1_ref[...], preferred_element_type=jnp.float32)
        h = jnp.maximum(h + q1_ref[...], 0.0)
        o = jnp.dot(h, p2_ref[...], preferred_element_type=jnp.float32)
        o = o + q2_ref[...]
        out_ref[...] = o[:, :2]

    full = lambda shape: pl.BlockSpec(shape, lambda i: (0, 0))
    return pl.pallas_call(
        body,
        grid=(N // _BN,),
        in_specs=[
            pl.BlockSpec((_BN, W), lambda i: (i, 0)),
            pl.BlockSpec((_BN, W), lambda i: (i, 0)),
            pl.BlockSpec((_BN, W), lambda i: (i, 0)),
            full(p0p.shape), full(p1.shape), full(p2p.shape),
            full(q0.shape), full(q1.shape), full(q2p.shape),
        ],
        out_specs=pl.BlockSpec((_BN, 2), lambda i: (i, 0)),
        out_shape=jax.ShapeDtypeStruct((N, 2), jnp.float32),
        interpret=interpret,
    )(a0, a1, tab, p0p, p1, p2p, q0, q1, q2p)


def kernel(pos, vel, field, index, edge_index, data_id, a,
           We0, We1, We2, We3, We4, be0, be1, be2, be3, be4,
           Wp0, Wp1, Wp2, bp0, bp1, bp2):
    f32 = jnp.float32
    N = pos.shape[0]
    dst = edge_index[0]
    src = edge_index[1]
    d_pos = (vel / _VNORM).astype(f32)
    emb = a[data_id, index]
    tab = jnp.concatenate(
        [pos.astype(f32), d_pos, emb.astype(f32),
         jnp.zeros((N, _TABW - 6), f32)], axis=1)

    gd, gs = _sc_gather(tab, dst, src)

    w0p = jnp.zeros((_TABW, We0.shape[1]), f32).at[:We0.shape[0]].set(We0)
    w4p = jnp.zeros((We4.shape[0], 128), f32).at[:, :We4.shape[1]].set(We4)
    b4p = jnp.zeros((1, 128), f32).at[0, :be4.shape[0]].set(be4)
    msg = _tc_edge_mlp(gd, gs, w0p, We1, We2, We3, w4p,
                       be0.reshape(1, -1), be1.reshape(1, -1),
                       be2.reshape(1, -1), be3.reshape(1, -1), b4p)

    acc0, acc1 = _sc_scatter(msg, dst, jnp.zeros((N, _TABW), f32))

    p0p = jnp.zeros((_TABW, Wp0.shape[1]), f32).at[:Wp0.shape[0]].set(Wp0)
    p2p = jnp.zeros((Wp2.shape[0], 128), f32).at[:, :Wp2.shape[1]].set(Wp2)
    q2p = jnp.zeros((1, 128), f32).at[0, :bp2.shape[0]].set(bp2)
    out = _tc_node_mlp(acc0, acc1, tab, p0p, Wp1, p2p,
                       bp0.reshape(1, -1), bp1.reshape(1, -1), q2p)
    return out


# speedup vs baseline: 6.5644x; 1.0320x over previous
import functools

import jax
import jax.numpy as jnp
from jax import lax
from jax.experimental import pallas as pl
from jax.experimental.pallas import tpu as pltpu
from jax.experimental.pallas import tpu_sc as plsc

_MAX_RADIUS = 0.1
_VNORM = 1.0
_NC = 2
_NS = 16
_NW = _NC * _NS
_TABW = 16
_GB = 1000
_CB = 1000
_BE = 3200
_BN = 1000


def _sc_gather(tab, dst, src, *, interpret=False):
    N, W = tab.shape
    E = dst.shape[0]
    ew = E // _NW
    nchunks = ew // _GB
    mesh = plsc.VectorSubcoreMesh(core_axis_name="c", subcore_axis_name="s")

    @functools.partial(
        pl.kernel,
        out_type=(jax.ShapeDtypeStruct((E, W), jnp.float32),
                  jax.ShapeDtypeStruct((E, W), jnp.float32)),
        mesh=mesh,
        scratch_types=[pltpu.VMEM((_GB,), jnp.int32),
                       pltpu.VMEM((_GB, W), jnp.float32),
                       pltpu.SemaphoreType.DMA],
        compiler_params=pltpu.CompilerParams(use_tc_tiling_on_sc=False),
        interpret=interpret,
    )
    def gk(tab_hbm, dst_hbm, src_hbm, gd_hbm, gs_hbm, idx_v, rows_v, sem):
        wid = lax.axis_index("s") * _NC + lax.axis_index("c")

        def body(t, carry):
            base = wid * ew + t * _GB
            pltpu.sync_copy(dst_hbm.at[pl.ds(base, _GB)], idx_v)
            pltpu.async_copy(tab_hbm.at[idx_v], rows_v, sem).wait()
            pltpu.sync_copy(rows_v, gd_hbm.at[pl.ds(base, _GB), :])
            pltpu.sync_copy(src_hbm.at[pl.ds(base, _GB)], idx_v)
            pltpu.async_copy(tab_hbm.at[idx_v], rows_v, sem).wait()
            pltpu.sync_copy(rows_v, gs_hbm.at[pl.ds(base, _GB), :])
            return carry

        lax.fori_loop(0, nchunks, body, 0)

    return gk(tab, dst, src)


def _sc_scatter(msg, dst, zero_tab, *, interpret=False):
    N, W = zero_tab.shape
    E = dst.shape[0]
    ew = E // _NW
    nchunks = ew // _CB
    rps = N // _NS
    mesh = plsc.VectorSubcoreMesh(core_axis_name="c", subcore_axis_name="s")

    @functools.partial(
        pl.kernel,
        out_type=(jax.ShapeDtypeStruct((N, W), jnp.float32),
                  jax.ShapeDtypeStruct((N, W), jnp.float32)),
        mesh=mesh,
        scratch_types=[pltpu.VMEM((_CB,), jnp.int32),
                       pltpu.VMEM((_CB, W), jnp.float32),
                       pltpu.VMEM_SHARED((N, W), jnp.float32)],
        compiler_params=pltpu.CompilerParams(use_tc_tiling_on_sc=False),
        interpret=interpret,
    )
    def sk(msg_hbm, dst_hbm, zero_hbm, a0_hbm, a1_hbm, idx_v, msg_v, acc_sh):
        cid = lax.axis_index("c")
        sid = lax.axis_index("s")
        wid = sid * _NC + cid

        @pl.when(sid == 0)
        def _zero():
            pltpu.sync_copy(zero_hbm, acc_sh)

        plsc.subcore_barrier()

        def body(t, carry):
            base = wid * ew + t * _CB
            pltpu.sync_copy(dst_hbm.at[pl.ds(base, _CB)], idx_v)
            pltpu.sync_copy(msg_hbm.at[pl.ds(base, _CB), :], msg_v)
            pltpu.sync_copy(msg_v, acc_sh.at[idx_v], add=True)
            return carry

        lax.fori_loop(0, nchunks, body, 0)
        plsc.subcore_barrier()
        row0 = sid * rps

        @pl.when(cid == 0)
        def _w0():
            pltpu.sync_copy(acc_sh.at[pl.ds(row0, rps), :],
                            a0_hbm.at[pl.ds(row0, rps), :])

        @pl.when(cid == 1)
        def _w1():
            pltpu.sync_copy(acc_sh.at[pl.ds(row0, rps), :],
                            a1_hbm.at[pl.ds(row0, rps), :])

    return sk(msg, dst, zero_tab)


def _tc_edge_mlp(gd, gs, w0p, w1, w2, w3, w4p, b0, b1, b2, b3, b4p,
                 *, interpret=False):
    E, W = gd.shape
    H = w1.shape[0]

    def body(gd_ref, gs_ref, w0_ref, w1_ref, w2_ref, w3_ref, w4_ref,
             b0_ref, b1_ref, b2_ref, b3_ref, b4_ref, out_ref):
        gd_b = gd_ref[...]
        gs_b = gs_ref[...]
        diff = gs_b[:, 0:2] - gd_b[:, 0:2]
        r = jnp.sqrt(jnp.sum(diff * diff, axis=1, keepdims=True) + 1e-12)
        scale = 1.0 / _MAX_RADIUS
        feats = jnp.concatenate(
            [diff * scale, r * scale, gd_b[:, 2:4], gs_b[:, 2:4],
             gd_b[:, 4:6], jnp.zeros((gd_b.shape[0], 7), jnp.float32)],
            axis=1)
        h = jnp.dot(feats, w0_ref[...], preferred_element_type=jnp.float32)
        h = jnp.maximum(h + b0_ref[...], 0.0)
        h = jnp.dot(h, w1_ref[...], preferred_element_type=jnp.float32)
        h = jnp.maximum(h + b1_ref[...], 0.0)
        h = jnp.dot(h, w2_ref[...], preferred_element_type=jnp.float32)
        h = jnp.maximum(h + b2_ref[...], 0.0)
        h = jnp.dot(h, w3_ref[...], preferred_element_type=jnp.float32)
        h = jnp.maximum(h + b3_ref[...], 0.0)
        m = jnp.dot(h, w4_ref[...], preferred_element_type=jnp.float32)
        m = m + b4_ref[...]
        out_ref[...] = m[:, :_TABW]

    full = lambda shape: pl.BlockSpec(shape, lambda i: (0, 0))
    return pl.pallas_call(
        body,
        grid=(E // _BE,),
        in_specs=[
            pl.BlockSpec((_BE, W), lambda i: (i, 0)),
            pl.BlockSpec((_BE, W), lambda i: (i, 0)),
            full(w0p.shape), full(w1.shape), full(w2.shape), full(w3.shape),
            full(w4p.shape),
            full(b0.shape), full(b1.shape), full(b2.shape), full(b3.shape),
            full(b4p.shape),
        ],
        out_specs=pl.BlockSpec((_BE, _TABW), lambda i: (i, 0)),
        out_shape=jax.ShapeDtypeStruct((E, _TABW), jnp.float32),
        interpret=interpret,
    )(gd, gs, w0p, w1, w2, w3, w4p, b0, b1, b2, b3, b4p)


def _tc_node_mlp(a0, a1, tab, p0p, p1, p2p, q0, q1, q2p, *, interpret=False):
    N, W = tab.shape

    def body(a0_ref, a1_ref, tab_ref, p0_ref, p1_ref, p2_ref,
             q0_ref, q1_ref, q2_ref, out_ref):
        agg = a0_ref[...][:, 0:2] + a1_ref[...][:, 0:2]
        t = tab_ref[...]
        feats = jnp.concatenate(
            [agg, t[:, 4:6], t[:, 2:4],
             jnp.zeros((agg.shape[0], 10), jnp.float32)], axis=1)
        h = jnp.dot(feats, p0_ref[...], preferred_element_type=jnp.float32)
        h = jnp.maximum(h + q0_ref[...], 0.0)
        h = jnp.dot(h, p1_ref[...], preferred_element_type=jnp.float32)
        h = jnp.maximum(h + q1_ref[...], 0.0)
        o = jnp.dot(h, p2_ref[...], preferred_element_type=jnp.float32)
        o = o + q2_ref[...]
        out_ref[...] = o[:, :2]

    full = lambda shape: pl.BlockSpec(shape, lambda i: (0, 0))
    return pl.pallas_call(
        body,
        grid=(N // _BN,),
        in_specs=[
            pl.BlockSpec((_BN, W), lambda i: (i, 0)),
            pl.BlockSpec((_BN, W), lambda i: (i, 0)),
            pl.BlockSpec((_BN, W), lambda i: (i, 0)),
            full(p0p.shape), full(p1.shape), full(p2p.shape),
            full(q0.shape), full(q1.shape), full(q2p.shape),
        ],
        out_specs=pl.BlockSpec((_BN, 2), lambda i: (i, 0)),
        out_shape=jax.ShapeDtypeStruct((N, 2), jnp.float32),
        interpret=interpret,
    )(a0, a1, tab, p0p, p1, p2p, q0, q1, q2p)


def kernel(pos, vel, field, index, edge_index, data_id, a,
           We0, We1, We2, We3, We4, be0, be1, be2, be3, be4,
           Wp0, Wp1, Wp2, bp0, bp1, bp2):
    f32 = jnp.float32
    N = pos.shape[0]
    dst = edge_index[0]
    src = edge_index[1]
    d_pos = (vel / _VNORM).astype(f32)
    emb = a[data_id, index]
    tab = jnp.concatenate(
        [pos.astype(f32), d_pos, emb.astype(f32),
         jnp.zeros((N, _TABW - 6), f32)], axis=1)

    gd, gs = _sc_gather(tab, dst, src)

    w0p = jnp.zeros((_TABW, We0.shape[1]), f32).at[:We0.shape[0]].set(We0)
    w4p = jnp.zeros((We4.shape[0], 128), f32).at[:, :We4.shape[1]].set(We4)
    b4p = jnp.zeros((1, 128), f32).at[0, :be4.shape[0]].set(be4)
    msg = _tc_edge_mlp(gd, gs, w0p, We1, We2, We3, w4p,
                       be0.reshape(1, -1), be1.reshape(1, -1),
                       be2.reshape(1, -1), be3.reshape(1, -1), b4p)

    acc0, acc1 = _sc_scatter(msg, dst, jnp.zeros((N, _TABW), f32))

    p0p = jnp.zeros((_TABW, Wp0.shape[1]), f32).at[:Wp0.shape[0]].set(Wp0)
    p2p = jnp.zeros((Wp2.shape[0], 128), f32).at[:, :Wp2.shape[1]].set(Wp2)
    q2p = jnp.zeros((1, 128), f32).at[0, :bp2.shape[0]].set(bp2)
    out = _tc_node_mlp(acc0, acc1, tab, p0p, Wp1, p2p,
                       bp0.reshape(1, -1), bp1.reshape(1, -1), q2p)
    return out
